# baseline (speedup 1.0000x reference)
import functools

import numpy as np
import jax
import jax.numpy as jnp
from jax import lax
from jax.experimental import pallas as pl
from jax.experimental.pallas import tpu as pltpu
from jax.experimental.pallas import tpu_sc as plsc

_NUM_TYPES = 8
_MAX_PARAMS = 16
_IN = 1024
_H1 = 512
_H2 = 256
_N = 32768
_NH = 2
_NT = _N // _NH
_B = 128
_GH = _NT // _B + _NUM_TYPES
_MPH = _GH * _B

_LANES = 128
_PCOUNTS = np.array([3, 4, 6, 7, 9, 10, 12, 16], dtype=np.int32)

try:
    _SC_INFO = plsc.get_sparse_core_info()
    _SC_NC, _SC_NS = _SC_INFO.num_cores, _SC_INFO.num_subcores
except Exception:
    _SC_NC, _SC_NS = 2, 16
_NW = _SC_NC * _SC_NS
_TPW = _NT // _NW
_DCH = 32
_CCH = 128


def _worker_id():
    return lax.axis_index("s") * _SC_NC + lax.axis_index("c")


def _dispatch_body(half, x_hbm, pos_hbm, xs_hbm, idx_v, rows_v, lsem, ssem):
    base = half * _NT + _worker_id() * _TPW
    pbase = _worker_id() * _TPW
    n = _TPW // _DCH
    loads = {}

    def load(j):
        b = j & 1
        loads[j] = (
            pltpu.async_copy(pos_hbm.at[pl.ds(pbase + j * _DCH, _DCH)],
                             idx_v.at[b], lsem),
            pltpu.async_copy(x_hbm.at[pl.ds(base + j * _DCH, _DCH)],
                             rows_v.at[b], lsem),
        )

    load(0)
    prev_scat = None
    for j in range(n):
        b = j & 1
        h1, h2 = loads.pop(j)
        h1.wait()
        h2.wait()
        if prev_scat is not None:
            prev_scat.wait()
        if j + 1 < n:
            load(j + 1)
        prev_scat = pltpu.async_copy(rows_v.at[b], xs_hbm.at[idx_v.at[b]], ssem)
    prev_scat.wait()


@functools.cache
def _make_dispatch(half):
    return pl.kernel(
        functools.partial(_dispatch_body, half),
        out_type=jax.ShapeDtypeStruct((_MPH, _IN), jnp.float32),
        mesh=plsc.VectorSubcoreMesh(core_axis_name="c", subcore_axis_name="s"),
        scratch_types=[
            pltpu.VMEM((2, _DCH), jnp.int32),
            pltpu.VMEM((2, _DCH, _IN), jnp.float32),
            pltpu.SemaphoreType.DMA,
            pltpu.SemaphoreType.DMA,
        ],
    )


def _collect_body(ps_hbm, pos_hbm, par_hbm, pidx_v, pbuf, isem, gsem, osem):
    base = _worker_id() * _TPW
    n = _TPW // _CCH
    idx_loads, gathers, writes = {}, {}, {}

    def load_idx(j):
        b = j & 1
        idx_loads[j] = pltpu.async_copy(
            pos_hbm.at[pl.ds(base + j * _CCH, _CCH)], pidx_v.at[b], isem)

    def start_gather(j):
        b = j & 1
        idx_loads.pop(j).wait()
        gathers[j] = pltpu.async_copy(ps_hbm.at[pidx_v.at[b]], pbuf.at[b], gsem)

    def start_write(j):
        b = j & 1
        gathers.pop(j).wait()
        writes[j] = pltpu.async_copy(
            pbuf.at[b], par_hbm.at[pl.ds(base + j * _CCH, _CCH)], osem)

    load_idx(0)
    start_gather(0)
    load_idx(1)
    for j in range(n):
        start_write(j)
        if j + 1 < n:
            if j - 1 in writes:
                writes.pop(j - 1).wait()
            start_gather(j + 1)
            if j + 2 < n:
                load_idx(j + 2)
    for j in sorted(writes):
        writes.pop(j).wait()


@functools.cache
def _make_collect():
    return pl.kernel(
        _collect_body,
        out_type=jax.ShapeDtypeStruct((_NT, _LANES), jnp.float32),
        mesh=plsc.VectorSubcoreMesh(core_axis_name="c", subcore_axis_name="s"),
        scratch_types=[
            pltpu.VMEM((2, _CCH), jnp.int32),
            pltpu.VMEM((2, _CCH, _LANES), jnp.float32),
            pltpu.SemaphoreType.DMA,
            pltpu.SemaphoreType.DMA,
            pltpu.SemaphoreType.DMA,
        ],
    )


_TR = _NT // 128
_TCOLS = 128


def _shift_lanes(v, k):
    return jnp.concatenate(
        [jnp.zeros(v.shape[:-1] + (k,), v.dtype), v[..., : v.shape[-1] - k]],
        axis=-1)


def _shift_rows(v, k):
    return jnp.concatenate(
        [jnp.zeros((k,) + v.shape[1:], v.dtype), v[: v.shape[0] - k]], axis=0)


def _route_body(t_ref, pos_ref, be_ref):
    t2 = t_ref[...]
    pres, rowcnts = [], []
    for e in range(_NUM_TYPES):
        pre = (t2 == e).astype(jnp.int32)
        k = 1
        while k < _TCOLS:
            pre = pre + _shift_lanes(pre, k)
            k *= 2
        pres.append(pre)
        rowcnts.append(pre[:, _TCOLS - 1:_TCOLS])
    cmat = jnp.concatenate(rowcnts, axis=1)
    s = cmat
    k = 1
    while k < _TR:
        s = s + _shift_rows(s, k)
        k *= 2
    rowbase = s - cmat
    counts = s[_TR - 1:_TR, :]
    blocksv = (counts + (_B - 1)) // _B
    segsz = blocksv * _B
    ss = segsz
    bl = blocksv
    for k in (1, 2, 4):
        ss = ss + _shift_lanes(ss, k)
        bl = bl + _shift_lanes(bl, k)
    seg_start = ss - segsz
    pos = jnp.zeros((_TR, _TCOLS), jnp.int32)
    for e in range(_NUM_TYPES):
        base_e = seg_start[:, e:e + 1] + rowbase[:, e:e + 1]
        pos = jnp.where(t2 == e, base_e + pres[e] - 1, pos)
    pos_ref[...] = pos
    g = lax.broadcasted_iota(jnp.int32, (8, _GH), 1)
    be = jnp.zeros((8, _GH), jnp.int32)
    for e in range(_NUM_TYPES):
        be = be + (bl[:, e:e + 1] <= g).astype(jnp.int32)
    be_ref[...] = jnp.minimum(be, _NUM_TYPES - 1)


def _route_kernel(t2):
    return pl.pallas_call(
        _route_body,
        out_shape=[
            jax.ShapeDtypeStruct((_TR, _TCOLS), jnp.int32),
            jax.ShapeDtypeStruct((8, _GH), jnp.int32),
        ],
    )(t2)


_MB = 4096


def _mask_body(t_ref, out_ref):
    tv = t_ref[...]
    pc = jnp.zeros_like(tv)
    for e in range(_NUM_TYPES):
        pc += jnp.where(tv == e, np.int32(_PCOUNTS[e]), 0)
    ik = lax.broadcasted_iota(jnp.int32, (_MB, _MAX_PARAMS), 1)
    out_ref[...] = (ik < pc[:, None]).astype(jnp.float32)


def _mask_kernel(t):
    return pl.pallas_call(
        _mask_body,
        grid=(_N // _MB,),
        in_specs=[pl.BlockSpec((_MB,), lambda i: (i,))],
        out_specs=pl.BlockSpec((_MB, _MAX_PARAMS), lambda i: (i, 0)),
        out_shape=jax.ShapeDtypeStruct((_N, _MAX_PARAMS), jnp.float32),
    )(t)


def _gelu_exact(v):
    return 0.5 * v * (1.0 + lax.erf(v * np.float32(1.0 / np.sqrt(2.0))))


def _mlp_body(be_ref, xs_ref, w1_ref, b1_ref, w2_ref, b2_ref, w3_ref, b3_ref,
              out_ref):
    x = xs_ref[...].astype(jnp.bfloat16)
    h = jnp.dot(x, w1_ref[0], preferred_element_type=jnp.float32) + b1_ref[0]
    h = _gelu_exact(h)
    h = jnp.dot(h, w2_ref[0], preferred_element_type=jnp.float32) + b2_ref[0]
    h = _gelu_exact(h)
    p = jnp.dot(h, w3_ref[0], preferred_element_type=jnp.float32) + b3_ref[0]
    out_ref[...] = jnp.concatenate(
        [p, jnp.zeros((_B, _LANES - _MAX_PARAMS), jnp.float32)], axis=1
    )


def _grouped_mlp(block_expert, xs, W1b, b1r, W2, b2r, W3, b3r):
    grid_spec = pltpu.PrefetchScalarGridSpec(
        num_scalar_prefetch=1,
        grid=(_GH,),
        in_specs=[
            pl.BlockSpec((_B, _IN), lambda i, be: (i, 0)),
            pl.BlockSpec((1, _IN, _H1), lambda i, be: (be[i], 0, 0)),
            pl.BlockSpec((1, 1, _H1), lambda i, be: (be[i], 0, 0)),
            pl.BlockSpec((1, _H1, _H2), lambda i, be: (be[i], 0, 0)),
            pl.BlockSpec((1, 1, _H2), lambda i, be: (be[i], 0, 0)),
            pl.BlockSpec((1, _H2, _MAX_PARAMS), lambda i, be: (be[i], 0, 0)),
            pl.BlockSpec((1, 1, _MAX_PARAMS), lambda i, be: (be[i], 0, 0)),
        ],
        out_specs=pl.BlockSpec((_B, _LANES), lambda i, be: (i, 0)),
    )
    return pl.pallas_call(
        _mlp_body,
        grid_spec=grid_spec,
        out_shape=jax.ShapeDtypeStruct((_MPH, _LANES), jnp.float32),
        compiler_params=pltpu.CompilerParams(
            dimension_semantics=("arbitrary",),
        ),
    )(block_expert, xs, W1b, b1r, W2, b2r, W3, b3r)


def kernel(x, geometry_types, W1, b1, W2, b2, W3, b3):
    t = geometry_types.astype(jnp.int32)
    W1b = W1.astype(jnp.bfloat16)
    b1r = b1.reshape(_NUM_TYPES, 1, _H1)
    b2r = b2.reshape(_NUM_TYPES, 1, _H2)
    b3r = b3.reshape(_NUM_TYPES, 1, _MAX_PARAMS)

    halves = []
    for h in range(_NH):
        th = lax.slice(t, (h * _NT,), ((h + 1) * _NT,))
        pos2, be2 = _route_kernel(th.reshape(_TR, _TCOLS))
        pos = pos2.reshape(_NT)
        xs = _make_dispatch(h)(x, pos)
        ps = _grouped_mlp(be2[0], xs, W1b, b1r, W2, b2r, W3, b3r)
        halves.append(_make_collect()(ps, pos))

    params = jnp.concatenate(
        [p[:, :_MAX_PARAMS] for p in halves], axis=0)
    masks = _mask_kernel(t)
    return params, masks


# speedup vs baseline: 1.4733x; 1.4733x over previous
import functools

import numpy as np
import jax
import jax.numpy as jnp
from jax import lax
from jax.experimental import pallas as pl
from jax.experimental.pallas import tpu as pltpu
from jax.experimental.pallas import tpu_sc as plsc

_NUM_TYPES = 8
_MAX_PARAMS = 16
_IN = 1024
_H1 = 512
_H2 = 256
_N = 32768
_NH = 2
_NT = _N // _NH
_B = 512
_GH = _NT // _B + _NUM_TYPES
_MPH = _GH * _B

_LANES = 128
_PCOUNTS = np.array([3, 4, 6, 7, 9, 10, 12, 16], dtype=np.int32)

try:
    _SC_INFO = plsc.get_sparse_core_info()
    _SC_NC, _SC_NS = _SC_INFO.num_cores, _SC_INFO.num_subcores
except Exception:
    _SC_NC, _SC_NS = 2, 16
_NW = _SC_NC * _SC_NS
_TPW = _NT // _NW
_DCH = 32
_CCH = 128


def _worker_id():
    return lax.axis_index("s") * _SC_NC + lax.axis_index("c")


def _dispatch_body(half, x_hbm, pos_hbm, xs_hbm, idx_v, rows_v, lsem, ssem):
    base = half * _NT + _worker_id() * _TPW
    pbase = _worker_id() * _TPW
    n = _TPW // _DCH
    loads = {}

    def load(j):
        b = j & 1
        loads[j] = (
            pltpu.async_copy(pos_hbm.at[pl.ds(pbase + j * _DCH, _DCH)],
                             idx_v.at[b], lsem),
            pltpu.async_copy(x_hbm.at[pl.ds(base + j * _DCH, _DCH)],
                             rows_v.at[b], lsem),
        )

    load(0)
    prev_scat = None
    for j in range(n):
        b = j & 1
        h1, h2 = loads.pop(j)
        h1.wait()
        h2.wait()
        if prev_scat is not None:
            prev_scat.wait()
        if j + 1 < n:
            load(j + 1)
        prev_scat = pltpu.async_copy(rows_v.at[b], xs_hbm.at[idx_v.at[b]], ssem)
    prev_scat.wait()


@functools.cache
def _make_dispatch(half):
    return pl.kernel(
        functools.partial(_dispatch_body, half),
        out_type=jax.ShapeDtypeStruct((_MPH, _IN), jnp.float32),
        mesh=plsc.VectorSubcoreMesh(core_axis_name="c", subcore_axis_name="s"),
        scratch_types=[
            pltpu.VMEM((2, _DCH), jnp.int32),
            pltpu.VMEM((2, _DCH, _IN), jnp.float32),
            pltpu.SemaphoreType.DMA,
            pltpu.SemaphoreType.DMA,
        ],
    )


def _collect_body(ps_hbm, pos_hbm, par_hbm, pidx_v, pbuf, isem, gsem, osem):
    base = _worker_id() * _TPW
    n = _TPW // _CCH
    idx_loads, gathers, writes = {}, {}, {}

    def load_idx(j):
        b = j & 1
        idx_loads[j] = pltpu.async_copy(
            pos_hbm.at[pl.ds(base + j * _CCH, _CCH)], pidx_v.at[b], isem)

    def start_gather(j):
        b = j & 1
        idx_loads.pop(j).wait()
        gathers[j] = pltpu.async_copy(ps_hbm.at[pidx_v.at[b]], pbuf.at[b], gsem)

    def start_write(j):
        b = j & 1
        gathers.pop(j).wait()
        writes[j] = pltpu.async_copy(
            pbuf.at[b], par_hbm.at[pl.ds(base + j * _CCH, _CCH)], osem)

    load_idx(0)
    start_gather(0)
    load_idx(1)
    for j in range(n):
        start_write(j)
        if j + 1 < n:
            if j - 1 in writes:
                writes.pop(j - 1).wait()
            start_gather(j + 1)
            if j + 2 < n:
                load_idx(j + 2)
    for j in sorted(writes):
        writes.pop(j).wait()


@functools.cache
def _make_collect():
    return pl.kernel(
        _collect_body,
        out_type=jax.ShapeDtypeStruct((_NT, _LANES), jnp.float32),
        mesh=plsc.VectorSubcoreMesh(core_axis_name="c", subcore_axis_name="s"),
        scratch_types=[
            pltpu.VMEM((2, _CCH), jnp.int32),
            pltpu.VMEM((2, _CCH, _LANES), jnp.float32),
            pltpu.SemaphoreType.DMA,
            pltpu.SemaphoreType.DMA,
            pltpu.SemaphoreType.DMA,
        ],
    )


_TR = _NT // 128
_TCOLS = 128


def _shift_lanes(v, k):
    return jnp.concatenate(
        [jnp.zeros(v.shape[:-1] + (k,), v.dtype), v[..., : v.shape[-1] - k]],
        axis=-1)


def _shift_rows(v, k):
    return jnp.concatenate(
        [jnp.zeros((k,) + v.shape[1:], v.dtype), v[: v.shape[0] - k]], axis=0)


def _route_body(t_ref, pos_ref, be_ref):
    t2 = t_ref[...]
    pres, rowcnts = [], []
    for e in range(_NUM_TYPES):
        pre = (t2 == e).astype(jnp.int32)
        k = 1
        while k < _TCOLS:
            pre = pre + _shift_lanes(pre, k)
            k *= 2
        pres.append(pre)
        rowcnts.append(pre[:, _TCOLS - 1:_TCOLS])
    cmat = jnp.concatenate(rowcnts, axis=1)
    s = cmat
    k = 1
    while k < _TR:
        s = s + _shift_rows(s, k)
        k *= 2
    rowbase = s - cmat
    counts = s[_TR - 1:_TR, :]
    blocksv = (counts + (_B - 1)) // _B
    segsz = blocksv * _B
    ss = segsz
    bl = blocksv
    for k in (1, 2, 4):
        ss = ss + _shift_lanes(ss, k)
        bl = bl + _shift_lanes(bl, k)
    seg_start = ss - segsz
    pos = jnp.zeros((_TR, _TCOLS), jnp.int32)
    for e in range(_NUM_TYPES):
        base_e = seg_start[:, e:e + 1] + rowbase[:, e:e + 1]
        pos = jnp.where(t2 == e, base_e + pres[e] - 1, pos)
    pos_ref[...] = pos
    g = lax.broadcasted_iota(jnp.int32, (8, _GH), 1)
    be = jnp.zeros((8, _GH), jnp.int32)
    for e in range(_NUM_TYPES):
        be = be + (bl[:, e:e + 1] <= g).astype(jnp.int32)
    be_ref[...] = jnp.minimum(be, _NUM_TYPES - 1)


def _route_kernel(t2):
    return pl.pallas_call(
        _route_body,
        out_shape=[
            jax.ShapeDtypeStruct((_TR, _TCOLS), jnp.int32),
            jax.ShapeDtypeStruct((8, _GH), jnp.int32),
        ],
    )(t2)


_MB = 4096


def _mask_body(t_ref, out_ref):
    tv = t_ref[...]
    pc = jnp.zeros_like(tv)
    for e in range(_NUM_TYPES):
        pc += jnp.where(tv == e, np.int32(_PCOUNTS[e]), 0)
    ik = lax.broadcasted_iota(jnp.int32, (_MB, _MAX_PARAMS), 1)
    out_ref[...] = (ik < pc[:, None]).astype(jnp.float32)


def _mask_kernel(t):
    return pl.pallas_call(
        _mask_body,
        grid=(_N // _MB,),
        in_specs=[pl.BlockSpec((_MB,), lambda i: (i,))],
        out_specs=pl.BlockSpec((_MB, _MAX_PARAMS), lambda i: (i, 0)),
        out_shape=jax.ShapeDtypeStruct((_N, _MAX_PARAMS), jnp.float32),
    )(t)


def _gelu_exact(v):
    return 0.5 * v * (1.0 + lax.erf(v * np.float32(1.0 / np.sqrt(2.0))))


def _mlp_body(be_ref, xs_ref, w1_ref, b1_ref, w2_ref, b2_ref, w3_ref, b3_ref,
              out_ref):
    x = xs_ref[...].astype(jnp.bfloat16)
    h = jnp.dot(x, w1_ref[0], preferred_element_type=jnp.float32) + b1_ref[0]
    h = _gelu_exact(h)
    h = jnp.dot(h, w2_ref[0], preferred_element_type=jnp.float32) + b2_ref[0]
    h = _gelu_exact(h)
    p = jnp.dot(h, w3_ref[0], preferred_element_type=jnp.float32) + b3_ref[0]
    out_ref[...] = jnp.concatenate(
        [p, jnp.zeros((_B, _LANES - _MAX_PARAMS), jnp.float32)], axis=1
    )


def _grouped_mlp(block_expert, xs, W1b, b1r, W2, b2r, W3, b3r):
    grid_spec = pltpu.PrefetchScalarGridSpec(
        num_scalar_prefetch=1,
        grid=(_GH,),
        in_specs=[
            pl.BlockSpec((_B, _IN), lambda i, be: (i, 0)),
            pl.BlockSpec((1, _IN, _H1), lambda i, be: (be[i], 0, 0)),
            pl.BlockSpec((1, 1, _H1), lambda i, be: (be[i], 0, 0)),
            pl.BlockSpec((1, _H1, _H2), lambda i, be: (be[i], 0, 0)),
            pl.BlockSpec((1, 1, _H2), lambda i, be: (be[i], 0, 0)),
            pl.BlockSpec((1, _H2, _MAX_PARAMS), lambda i, be: (be[i], 0, 0)),
            pl.BlockSpec((1, 1, _MAX_PARAMS), lambda i, be: (be[i], 0, 0)),
        ],
        out_specs=pl.BlockSpec((_B, _LANES), lambda i, be: (i, 0)),
    )
    return pl.pallas_call(
        _mlp_body,
        grid_spec=grid_spec,
        out_shape=jax.ShapeDtypeStruct((_MPH, _LANES), jnp.float32),
        compiler_params=pltpu.CompilerParams(
            dimension_semantics=("arbitrary",),
        ),
    )(block_expert, xs, W1b, b1r, W2, b2r, W3, b3r)


def kernel(x, geometry_types, W1, b1, W2, b2, W3, b3):
    t = geometry_types.astype(jnp.int32)
    W1b = W1.astype(jnp.bfloat16)
    b1r = b1.reshape(_NUM_TYPES, 1, _H1)
    b2r = b2.reshape(_NUM_TYPES, 1, _H2)
    b3r = b3.reshape(_NUM_TYPES, 1, _MAX_PARAMS)

    halves = []
    for h in range(_NH):
        th = lax.slice(t, (h * _NT,), ((h + 1) * _NT,))
        pos2, be2 = _route_kernel(th.reshape(_TR, _TCOLS))
        pos = pos2.reshape(_NT)
        xs = _make_dispatch(h)(x, pos)
        ps = _grouped_mlp(be2[0], xs, W1b, b1r, W2, b2r, W3, b3r)
        halves.append(_make_collect()(ps, pos))

    params = jnp.concatenate(
        [p[:, :_MAX_PARAMS] for p in halves], axis=0)
    masks = _mask_kernel(t)
    return params, masks


# speedup vs baseline: 1.4952x; 1.0148x over previous
import functools

import numpy as np
import jax
import jax.numpy as jnp
from jax import lax
from jax.experimental import pallas as pl
from jax.experimental.pallas import tpu as pltpu
from jax.experimental.pallas import tpu_sc as plsc

_NUM_TYPES = 8
_MAX_PARAMS = 16
_IN = 1024
_H1 = 512
_H2 = 256
_N = 32768
_NH = 1
_NT = _N // _NH
_B = 512
_GH = _NT // _B + _NUM_TYPES
_MPH = _GH * _B

_LANES = 128
_PCOUNTS = np.array([3, 4, 6, 7, 9, 10, 12, 16], dtype=np.int32)

try:
    _SC_INFO = plsc.get_sparse_core_info()
    _SC_NC, _SC_NS = _SC_INFO.num_cores, _SC_INFO.num_subcores
except Exception:
    _SC_NC, _SC_NS = 2, 16
_NW = _SC_NC * _SC_NS
_TPW = _NT // _NW
_DCH = 32
_CCH = 128


def _worker_id():
    return lax.axis_index("s") * _SC_NC + lax.axis_index("c")


def _dispatch_body(half, x_hbm, pos_hbm, xs_hbm, idx_v, rows_v, lsem, ssem):
    base = half * _NT + _worker_id() * _TPW
    pbase = _worker_id() * _TPW
    n = _TPW // _DCH
    loads = {}

    def load(j):
        b = j & 1
        loads[j] = (
            pltpu.async_copy(pos_hbm.at[pl.ds(pbase + j * _DCH, _DCH)],
                             idx_v.at[b], lsem),
            pltpu.async_copy(x_hbm.at[pl.ds(base + j * _DCH, _DCH)],
                             rows_v.at[b], lsem),
        )

    load(0)
    prev_scat = None
    for j in range(n):
        b = j & 1
        h1, h2 = loads.pop(j)
        h1.wait()
        h2.wait()
        if prev_scat is not None:
            prev_scat.wait()
        if j + 1 < n:
            load(j + 1)
        prev_scat = pltpu.async_copy(rows_v.at[b], xs_hbm.at[idx_v.at[b]], ssem)
    prev_scat.wait()


@functools.cache
def _make_dispatch(half):
    return pl.kernel(
        functools.partial(_dispatch_body, half),
        out_type=jax.ShapeDtypeStruct((_MPH, _IN), jnp.float32),
        mesh=plsc.VectorSubcoreMesh(core_axis_name="c", subcore_axis_name="s"),
        scratch_types=[
            pltpu.VMEM((2, _DCH), jnp.int32),
            pltpu.VMEM((2, _DCH, _IN), jnp.float32),
            pltpu.SemaphoreType.DMA,
            pltpu.SemaphoreType.DMA,
        ],
    )


def _collect_body(ps_hbm, pos_hbm, par_hbm, pidx_v, pbuf, isem, gsem, osem):
    base = _worker_id() * _TPW
    n = _TPW // _CCH
    idx_loads, gathers, writes = {}, {}, {}

    def load_idx(j):
        b = j & 1
        idx_loads[j] = pltpu.async_copy(
            pos_hbm.at[pl.ds(base + j * _CCH, _CCH)], pidx_v.at[b], isem)

    def start_gather(j):
        b = j & 1
        idx_loads.pop(j).wait()
        gathers[j] = pltpu.async_copy(ps_hbm.at[pidx_v.at[b]], pbuf.at[b], gsem)

    def start_write(j):
        b = j & 1
        gathers.pop(j).wait()
        writes[j] = pltpu.async_copy(
            pbuf.at[b], par_hbm.at[pl.ds(base + j * _CCH, _CCH)], osem)

    load_idx(0)
    start_gather(0)
    load_idx(1)
    for j in range(n):
        start_write(j)
        if j + 1 < n:
            if j - 1 in writes:
                writes.pop(j - 1).wait()
            start_gather(j + 1)
            if j + 2 < n:
                load_idx(j + 2)
    for j in sorted(writes):
        writes.pop(j).wait()


@functools.cache
def _make_collect():
    return pl.kernel(
        _collect_body,
        out_type=jax.ShapeDtypeStruct((_NT, _LANES), jnp.float32),
        mesh=plsc.VectorSubcoreMesh(core_axis_name="c", subcore_axis_name="s"),
        scratch_types=[
            pltpu.VMEM((2, _CCH), jnp.int32),
            pltpu.VMEM((2, _CCH, _LANES), jnp.float32),
            pltpu.SemaphoreType.DMA,
            pltpu.SemaphoreType.DMA,
            pltpu.SemaphoreType.DMA,
        ],
    )


_TR = _NT // 128
_TCOLS = 128


def _shift_lanes(v, k):
    return jnp.concatenate(
        [jnp.zeros(v.shape[:-1] + (k,), v.dtype), v[..., : v.shape[-1] - k]],
        axis=-1)


def _shift_rows(v, k):
    return jnp.concatenate(
        [jnp.zeros((k,) + v.shape[1:], v.dtype), v[: v.shape[0] - k]], axis=0)


def _route_body(t_ref, pos_ref, be_ref):
    t2 = t_ref[...]
    pres, rowcnts = [], []
    for e in range(_NUM_TYPES):
        pre = (t2 == e).astype(jnp.int32)
        k = 1
        while k < _TCOLS:
            pre = pre + _shift_lanes(pre, k)
            k *= 2
        pres.append(pre)
        rowcnts.append(pre[:, _TCOLS - 1:_TCOLS])
    cmat = jnp.concatenate(rowcnts, axis=1)
    s = cmat
    k = 1
    while k < _TR:
        s = s + _shift_rows(s, k)
        k *= 2
    rowbase = s - cmat
    counts = s[_TR - 1:_TR, :]
    blocksv = (counts + (_B - 1)) // _B
    segsz = blocksv * _B
    ss = segsz
    bl = blocksv
    for k in (1, 2, 4):
        ss = ss + _shift_lanes(ss, k)
        bl = bl + _shift_lanes(bl, k)
    seg_start = ss - segsz
    pos = jnp.zeros((_TR, _TCOLS), jnp.int32)
    for e in range(_NUM_TYPES):
        base_e = seg_start[:, e:e + 1] + rowbase[:, e:e + 1]
        pos = jnp.where(t2 == e, base_e + pres[e] - 1, pos)
    pos_ref[...] = pos
    g = lax.broadcasted_iota(jnp.int32, (8, _GH), 1)
    be = jnp.zeros((8, _GH), jnp.int32)
    for e in range(_NUM_TYPES):
        be = be + (bl[:, e:e + 1] <= g).astype(jnp.int32)
    be_ref[...] = jnp.minimum(be, _NUM_TYPES - 1)


def _route_kernel(t2):
    return pl.pallas_call(
        _route_body,
        out_shape=[
            jax.ShapeDtypeStruct((_TR, _TCOLS), jnp.int32),
            jax.ShapeDtypeStruct((8, _GH), jnp.int32),
        ],
    )(t2)


_MB = 4096


def _mask_body(t_ref, out_ref):
    tv = t_ref[...]
    pc = jnp.zeros_like(tv)
    for e in range(_NUM_TYPES):
        pc += jnp.where(tv == e, np.int32(_PCOUNTS[e]), 0)
    ik = lax.broadcasted_iota(jnp.int32, (_MB, _MAX_PARAMS), 1)
    out_ref[...] = (ik < pc[:, None]).astype(jnp.float32)


def _mask_kernel(t):
    return pl.pallas_call(
        _mask_body,
        grid=(_N // _MB,),
        in_specs=[pl.BlockSpec((_MB,), lambda i: (i,))],
        out_specs=pl.BlockSpec((_MB, _MAX_PARAMS), lambda i: (i, 0)),
        out_shape=jax.ShapeDtypeStruct((_N, _MAX_PARAMS), jnp.float32),
    )(t)


def _gelu_exact(v):
    return 0.5 * v * (1.0 + lax.erf(v * np.float32(1.0 / np.sqrt(2.0))))


def _mlp_body(be_ref, xs_ref, w1_ref, b1_ref, w2_ref, b2_ref, w3_ref, b3_ref,
              out_ref):
    x = xs_ref[...].astype(jnp.bfloat16)
    h = jnp.dot(x, w1_ref[0], preferred_element_type=jnp.float32) + b1_ref[0]
    h = _gelu_exact(h)
    h = jnp.dot(h, w2_ref[0], preferred_element_type=jnp.float32) + b2_ref[0]
    h = _gelu_exact(h)
    p = jnp.dot(h, w3_ref[0], preferred_element_type=jnp.float32) + b3_ref[0]
    out_ref[...] = jnp.concatenate(
        [p, jnp.zeros((_B, _LANES - _MAX_PARAMS), jnp.float32)], axis=1
    )


def _grouped_mlp(block_expert, xs, W1b, b1r, W2, b2r, W3, b3r):
    grid_spec = pltpu.PrefetchScalarGridSpec(
        num_scalar_prefetch=1,
        grid=(_GH,),
        in_specs=[
            pl.BlockSpec((_B, _IN), lambda i, be: (i, 0)),
            pl.BlockSpec((1, _IN, _H1), lambda i, be: (be[i], 0, 0)),
            pl.BlockSpec((1, 1, _H1), lambda i, be: (be[i], 0, 0)),
            pl.BlockSpec((1, _H1, _H2), lambda i, be: (be[i], 0, 0)),
            pl.BlockSpec((1, 1, _H2), lambda i, be: (be[i], 0, 0)),
            pl.BlockSpec((1, _H2, _MAX_PARAMS), lambda i, be: (be[i], 0, 0)),
            pl.BlockSpec((1, 1, _MAX_PARAMS), lambda i, be: (be[i], 0, 0)),
        ],
        out_specs=pl.BlockSpec((_B, _LANES), lambda i, be: (i, 0)),
    )
    return pl.pallas_call(
        _mlp_body,
        grid_spec=grid_spec,
        out_shape=jax.ShapeDtypeStruct((_MPH, _LANES), jnp.float32),
        compiler_params=pltpu.CompilerParams(
            dimension_semantics=("arbitrary",),
        ),
    )(block_expert, xs, W1b, b1r, W2, b2r, W3, b3r)


def kernel(x, geometry_types, W1, b1, W2, b2, W3, b3):
    t = geometry_types.astype(jnp.int32)
    W1b = W1.astype(jnp.bfloat16)
    b1r = b1.reshape(_NUM_TYPES, 1, _H1)
    b2r = b2.reshape(_NUM_TYPES, 1, _H2)
    b3r = b3.reshape(_NUM_TYPES, 1, _MAX_PARAMS)

    halves = []
    for h in range(_NH):
        th = lax.slice(t, (h * _NT,), ((h + 1) * _NT,))
        pos2, be2 = _route_kernel(th.reshape(_TR, _TCOLS))
        pos = pos2.reshape(_NT)
        xs = _make_dispatch(h)(x, pos)
        ps = _grouped_mlp(be2[0], xs, W1b, b1r, W2, b2r, W3, b3r)
        halves.append(_make_collect()(ps, pos))

    params = jnp.concatenate(
        [p[:, :_MAX_PARAMS] for p in halves], axis=0)
    masks = _mask_kernel(t)
    return params, masks


# speedup vs baseline: 1.5002x; 1.0034x over previous
import functools

import numpy as np
import jax
import jax.numpy as jnp
from jax import lax
from jax.experimental import pallas as pl
from jax.experimental.pallas import tpu as pltpu
from jax.experimental.pallas import tpu_sc as plsc

_NUM_TYPES = 8
_MAX_PARAMS = 16
_IN = 1024
_H1 = 512
_H2 = 256
_N = 32768
_NH = 1
_NT = _N // _NH
_B = 512
_GH = _NT // _B + _NUM_TYPES
_MPH = _GH * _B

_LANES = 128
_PCOUNTS = np.array([3, 4, 6, 7, 9, 10, 12, 16], dtype=np.int32)

try:
    _SC_INFO = plsc.get_sparse_core_info()
    _SC_NC, _SC_NS = _SC_INFO.num_cores, _SC_INFO.num_subcores
except Exception:
    _SC_NC, _SC_NS = 2, 16
_NW = _SC_NC * _SC_NS
_TPW = _NT // _NW
_DCH = 32
_CCH = 128


def _worker_id():
    return lax.axis_index("s") * _SC_NC + lax.axis_index("c")


def _dispatch_body(half, x_hbm, pos_hbm, xs_hbm, idx_v, rows_v, lsem, ssem):
    base = half * _NT + _worker_id() * _TPW
    pbase = _worker_id() * _TPW
    n = _TPW // _DCH
    loads = {}

    def load(j):
        b = j & 1
        loads[j] = (
            pltpu.async_copy(pos_hbm.at[pl.ds(pbase + j * _DCH, _DCH)],
                             idx_v.at[b], lsem),
            pltpu.async_copy(x_hbm.at[pl.ds(base + j * _DCH, _DCH)],
                             rows_v.at[b], lsem),
        )

    load(0)
    prev_scat = None
    for j in range(n):
        b = j & 1
        h1, h2 = loads.pop(j)
        h1.wait()
        h2.wait()
        if prev_scat is not None:
            prev_scat.wait()
        if j + 1 < n:
            load(j + 1)
        prev_scat = pltpu.async_copy(rows_v.at[b], xs_hbm.at[idx_v.at[b]], ssem)
    prev_scat.wait()


@functools.cache
def _make_dispatch(half):
    return pl.kernel(
        functools.partial(_dispatch_body, half),
        out_type=jax.ShapeDtypeStruct((_MPH, _IN), jnp.float32),
        mesh=plsc.VectorSubcoreMesh(core_axis_name="c", subcore_axis_name="s"),
        scratch_types=[
            pltpu.VMEM((2, _DCH), jnp.int32),
            pltpu.VMEM((2, _DCH, _IN), jnp.float32),
            pltpu.SemaphoreType.DMA,
            pltpu.SemaphoreType.DMA,
        ],
    )


def _collect_body(ps_hbm, pos_hbm, par_hbm, pidx_v, pbuf, isem, gsem, osem):
    base = _worker_id() * _TPW
    n = _TPW // _CCH
    idx_loads, gathers, writes = {}, {}, {}

    def load_idx(j):
        b = j & 1
        idx_loads[j] = pltpu.async_copy(
            pos_hbm.at[pl.ds(base + j * _CCH, _CCH)], pidx_v.at[b], isem)

    def start_gather(j):
        b = j & 1
        idx_loads.pop(j).wait()
        gathers[j] = pltpu.async_copy(ps_hbm.at[pidx_v.at[b]], pbuf.at[b], gsem)

    def start_write(j):
        b = j & 1
        gathers.pop(j).wait()
        writes[j] = pltpu.async_copy(
            pbuf.at[b], par_hbm.at[pl.ds(base + j * _CCH, _CCH)], osem)

    load_idx(0)
    start_gather(0)
    load_idx(1)
    for j in range(n):
        start_write(j)
        if j + 1 < n:
            if j - 1 in writes:
                writes.pop(j - 1).wait()
            start_gather(j + 1)
            if j + 2 < n:
                load_idx(j + 2)
    for j in sorted(writes):
        writes.pop(j).wait()


@functools.cache
def _make_collect():
    return pl.kernel(
        _collect_body,
        out_type=jax.ShapeDtypeStruct((_NT, _LANES), jnp.float32),
        mesh=plsc.VectorSubcoreMesh(core_axis_name="c", subcore_axis_name="s"),
        scratch_types=[
            pltpu.VMEM((2, _CCH), jnp.int32),
            pltpu.VMEM((2, _CCH, _LANES), jnp.float32),
            pltpu.SemaphoreType.DMA,
            pltpu.SemaphoreType.DMA,
            pltpu.SemaphoreType.DMA,
        ],
    )


_TR = _NT // 128
_TCOLS = 128


def _shift_lanes(v, k):
    return jnp.concatenate(
        [jnp.zeros(v.shape[:-1] + (k,), v.dtype), v[..., : v.shape[-1] - k]],
        axis=-1)


def _shift_rows(v, k):
    return jnp.concatenate(
        [jnp.zeros((k,) + v.shape[1:], v.dtype), v[: v.shape[0] - k]], axis=0)


def _route_body(t_ref, pos_ref, be_ref):
    t2 = t_ref[...]
    pres, rowcnts = [], []
    for e in range(_NUM_TYPES):
        pre = (t2 == e).astype(jnp.int32)
        k = 1
        while k < _TCOLS:
            pre = pre + _shift_lanes(pre, k)
            k *= 2
        pres.append(pre)
        rowcnts.append(pre[:, _TCOLS - 1:_TCOLS])
    cmat = jnp.concatenate(rowcnts, axis=1)
    s = cmat
    k = 1
    while k < _TR:
        s = s + _shift_rows(s, k)
        k *= 2
    rowbase = s - cmat
    counts = s[_TR - 1:_TR, :]
    blocksv = (counts + (_B - 1)) // _B
    segsz = blocksv * _B
    ss = segsz
    bl = blocksv
    for k in (1, 2, 4):
        ss = ss + _shift_lanes(ss, k)
        bl = bl + _shift_lanes(bl, k)
    seg_start = ss - segsz
    pos = jnp.zeros((_TR, _TCOLS), jnp.int32)
    for e in range(_NUM_TYPES):
        base_e = seg_start[:, e:e + 1] + rowbase[:, e:e + 1]
        pos = jnp.where(t2 == e, base_e + pres[e] - 1, pos)
    pos_ref[...] = pos
    g = lax.broadcasted_iota(jnp.int32, (8, _GH), 1)
    be = jnp.zeros((8, _GH), jnp.int32)
    for e in range(_NUM_TYPES):
        be = be + (bl[:, e:e + 1] <= g).astype(jnp.int32)
    be_ref[...] = jnp.minimum(be, _NUM_TYPES - 1)


def _route_kernel(t2):
    return pl.pallas_call(
        _route_body,
        out_shape=[
            jax.ShapeDtypeStruct((_TR, _TCOLS), jnp.int32),
            jax.ShapeDtypeStruct((8, _GH), jnp.int32),
        ],
    )(t2)


_MB = 4096


def _mask_body(t_ref, out_ref):
    tv = t_ref[...]
    pc = jnp.zeros_like(tv)
    for e in range(_NUM_TYPES):
        pc += jnp.where(tv == e, np.int32(_PCOUNTS[e]), 0)
    ik = lax.broadcasted_iota(jnp.int32, (_MB, _MAX_PARAMS), 1)
    out_ref[...] = (ik < pc[:, None]).astype(jnp.float32)


def _mask_kernel(t):
    return pl.pallas_call(
        _mask_body,
        grid=(_N // _MB,),
        in_specs=[pl.BlockSpec((_MB,), lambda i: (i,))],
        out_specs=pl.BlockSpec((_MB, _MAX_PARAMS), lambda i: (i, 0)),
        out_shape=jax.ShapeDtypeStruct((_N, _MAX_PARAMS), jnp.float32),
    )(t)


def _gelu_exact(v):
    return 0.5 * v * (1.0 + lax.erf(v * np.float32(1.0 / np.sqrt(2.0))))


def _mlp_body(be_ref, xs_ref, w1_ref, b1_ref, w2_ref, b2_ref, w3_ref, b3_ref,
              out_ref):
    x = xs_ref[...].astype(jnp.bfloat16)
    h = jnp.dot(x, w1_ref[0], preferred_element_type=jnp.float32) + b1_ref[0]
    h = _gelu_exact(h)
    h = jnp.dot(h, w2_ref[0], preferred_element_type=jnp.float32) + b2_ref[0]
    h = _gelu_exact(h)
    p = jnp.dot(h, w3_ref[0], preferred_element_type=jnp.float32) + b3_ref[0]
    out_ref[:, :_MAX_PARAMS] = p


def _grouped_mlp(block_expert, xs, W1b, b1r, W2, b2r, W3, b3r):
    grid_spec = pltpu.PrefetchScalarGridSpec(
        num_scalar_prefetch=1,
        grid=(_GH,),
        in_specs=[
            pl.BlockSpec((_B, _IN), lambda i, be: (i, 0)),
            pl.BlockSpec((1, _IN, _H1), lambda i, be: (be[i], 0, 0)),
            pl.BlockSpec((1, 1, _H1), lambda i, be: (be[i], 0, 0)),
            pl.BlockSpec((1, _H1, _H2), lambda i, be: (be[i], 0, 0)),
            pl.BlockSpec((1, 1, _H2), lambda i, be: (be[i], 0, 0)),
            pl.BlockSpec((1, _H2, _MAX_PARAMS), lambda i, be: (be[i], 0, 0)),
            pl.BlockSpec((1, 1, _MAX_PARAMS), lambda i, be: (be[i], 0, 0)),
        ],
        out_specs=pl.BlockSpec((_B, _LANES), lambda i, be: (i, 0)),
    )
    return pl.pallas_call(
        _mlp_body,
        grid_spec=grid_spec,
        out_shape=jax.ShapeDtypeStruct((_MPH, _LANES), jnp.float32),
        compiler_params=pltpu.CompilerParams(
            dimension_semantics=("arbitrary",),
        ),
    )(block_expert, xs, W1b, b1r, W2, b2r, W3, b3r)


def kernel(x, geometry_types, W1, b1, W2, b2, W3, b3):
    t = geometry_types.astype(jnp.int32)
    W1b = W1.astype(jnp.bfloat16)
    b1r = b1.reshape(_NUM_TYPES, 1, _H1)
    b2r = b2.reshape(_NUM_TYPES, 1, _H2)
    b3r = b3.reshape(_NUM_TYPES, 1, _MAX_PARAMS)

    halves = []
    for h in range(_NH):
        th = lax.slice(t, (h * _NT,), ((h + 1) * _NT,))
        pos2, be2 = _route_kernel(th.reshape(_TR, _TCOLS))
        pos = pos2.reshape(_NT)
        xs = _make_dispatch(h)(x, pos)
        ps = _grouped_mlp(be2[0], xs, W1b, b1r, W2, b2r, W3, b3r)
        halves.append(_make_collect()(ps, pos))

    params = jnp.concatenate(
        [p[:, :_MAX_PARAMS] for p in halves], axis=0)
    masks = _mask_kernel(t)
    return params, masks


# speedup vs baseline: 1.5068x; 1.0044x over previous
import functools

import numpy as np
import jax
import jax.numpy as jnp
from jax import lax
from jax.experimental import pallas as pl
from jax.experimental.pallas import tpu as pltpu
from jax.experimental.pallas import tpu_sc as plsc

_NUM_TYPES = 8
_MAX_PARAMS = 16
_IN = 1024
_H1 = 512
_H2 = 256
_N = 32768
_NH = 1
_NT = _N // _NH
_B = 512
_GH = _NT // _B + _NUM_TYPES
_MPH = _GH * _B

_LANES = 128
_PCOUNTS = np.array([3, 4, 6, 7, 9, 10, 12, 16], dtype=np.int32)

try:
    _SC_INFO = plsc.get_sparse_core_info()
    _SC_NC, _SC_NS = _SC_INFO.num_cores, _SC_INFO.num_subcores
except Exception:
    _SC_NC, _SC_NS = 2, 16
_NW = _SC_NC * _SC_NS
_TPW = _NT // _NW
_DCH = 32
_CCH = 128


def _worker_id():
    return lax.axis_index("s") * _SC_NC + lax.axis_index("c")


def _dispatch_body(half, x_hbm, pos_hbm, xs_hbm, idx_v, rows_v, lsem, ssem):
    base = half * _NT + _worker_id() * _TPW
    pbase = _worker_id() * _TPW
    n = _TPW // _DCH
    loads, scats = {}, {}

    def load(j):
        b = j % 3
        loads[j] = (
            pltpu.async_copy(pos_hbm.at[pl.ds(pbase + j * _DCH, _DCH)],
                             idx_v.at[b], lsem),
            pltpu.async_copy(x_hbm.at[pl.ds(base + j * _DCH, _DCH)],
                             rows_v.at[b], lsem),
        )

    load(0)
    load(1)
    for j in range(n):
        b = j % 3
        h1, h2 = loads.pop(j)
        h1.wait()
        h2.wait()
        scats[j] = pltpu.async_copy(rows_v.at[b], xs_hbm.at[idx_v.at[b]], ssem)
        if j + 2 < n:
            if j - 1 in scats:
                scats.pop(j - 1).wait()
            load(j + 2)
    for j in sorted(scats):
        scats.pop(j).wait()


@functools.cache
def _make_dispatch(half):
    return pl.kernel(
        functools.partial(_dispatch_body, half),
        out_type=jax.ShapeDtypeStruct((_MPH, _IN), jnp.float32),
        mesh=plsc.VectorSubcoreMesh(core_axis_name="c", subcore_axis_name="s"),
        scratch_types=[
            pltpu.VMEM((3, _DCH), jnp.int32),
            pltpu.VMEM((3, _DCH, _IN), jnp.float32),
            pltpu.SemaphoreType.DMA,
            pltpu.SemaphoreType.DMA,
        ],
    )


def _collect_body(ps_hbm, pos_hbm, par_hbm, pidx_v, pbuf, isem, gsem, osem):
    base = _worker_id() * _TPW
    n = _TPW // _CCH
    idx_loads, gathers, writes = {}, {}, {}

    def load_idx(j):
        b = j & 1
        idx_loads[j] = pltpu.async_copy(
            pos_hbm.at[pl.ds(base + j * _CCH, _CCH)], pidx_v.at[b], isem)

    def start_gather(j):
        b = j & 1
        idx_loads.pop(j).wait()
        gathers[j] = pltpu.async_copy(ps_hbm.at[pidx_v.at[b]], pbuf.at[b], gsem)

    def start_write(j):
        b = j & 1
        gathers.pop(j).wait()
        writes[j] = pltpu.async_copy(
            pbuf.at[b], par_hbm.at[pl.ds(base + j * _CCH, _CCH)], osem)

    load_idx(0)
    start_gather(0)
    load_idx(1)
    for j in range(n):
        start_write(j)
        if j + 1 < n:
            if j - 1 in writes:
                writes.pop(j - 1).wait()
            start_gather(j + 1)
            if j + 2 < n:
                load_idx(j + 2)
    for j in sorted(writes):
        writes.pop(j).wait()


@functools.cache
def _make_collect():
    return pl.kernel(
        _collect_body,
        out_type=jax.ShapeDtypeStruct((_NT, _LANES), jnp.float32),
        mesh=plsc.VectorSubcoreMesh(core_axis_name="c", subcore_axis_name="s"),
        scratch_types=[
            pltpu.VMEM((2, _CCH), jnp.int32),
            pltpu.VMEM((2, _CCH, _LANES), jnp.float32),
            pltpu.SemaphoreType.DMA,
            pltpu.SemaphoreType.DMA,
            pltpu.SemaphoreType.DMA,
        ],
    )


_TR = _NT // 128
_TCOLS = 128


def _shift_lanes(v, k):
    return jnp.concatenate(
        [jnp.zeros(v.shape[:-1] + (k,), v.dtype), v[..., : v.shape[-1] - k]],
        axis=-1)


def _shift_rows(v, k):
    return jnp.concatenate(
        [jnp.zeros((k,) + v.shape[1:], v.dtype), v[: v.shape[0] - k]], axis=0)


def _route_body(t_ref, pos_ref, be_ref):
    t2 = t_ref[...]
    pres, rowcnts = [], []
    for e in range(_NUM_TYPES):
        pre = (t2 == e).astype(jnp.int32)
        k = 1
        while k < _TCOLS:
            pre = pre + _shift_lanes(pre, k)
            k *= 2
        pres.append(pre)
        rowcnts.append(pre[:, _TCOLS - 1:_TCOLS])
    cmat = jnp.concatenate(rowcnts, axis=1)
    s = cmat
    k = 1
    while k < _TR:
        s = s + _shift_rows(s, k)
        k *= 2
    rowbase = s - cmat
    counts = s[_TR - 1:_TR, :]
    blocksv = (counts + (_B - 1)) // _B
    segsz = blocksv * _B
    ss = segsz
    bl = blocksv
    for k in (1, 2, 4):
        ss = ss + _shift_lanes(ss, k)
        bl = bl + _shift_lanes(bl, k)
    seg_start = ss - segsz
    pos = jnp.zeros((_TR, _TCOLS), jnp.int32)
    for e in range(_NUM_TYPES):
        base_e = seg_start[:, e:e + 1] + rowbase[:, e:e + 1]
        pos = jnp.where(t2 == e, base_e + pres[e] - 1, pos)
    pos_ref[...] = pos
    g = lax.broadcasted_iota(jnp.int32, (8, _GH), 1)
    be = jnp.zeros((8, _GH), jnp.int32)
    for e in range(_NUM_TYPES):
        be = be + (bl[:, e:e + 1] <= g).astype(jnp.int32)
    be_ref[...] = jnp.minimum(be, _NUM_TYPES - 1)


def _route_kernel(t2):
    return pl.pallas_call(
        _route_body,
        out_shape=[
            jax.ShapeDtypeStruct((_TR, _TCOLS), jnp.int32),
            jax.ShapeDtypeStruct((8, _GH), jnp.int32),
        ],
    )(t2)


_MB = 4096


def _mask_body(t_ref, out_ref):
    tv = t_ref[...]
    pc = jnp.zeros_like(tv)
    for e in range(_NUM_TYPES):
        pc += jnp.where(tv == e, np.int32(_PCOUNTS[e]), 0)
    ik = lax.broadcasted_iota(jnp.int32, (_MB, _MAX_PARAMS), 1)
    out_ref[...] = (ik < pc[:, None]).astype(jnp.float32)


def _mask_kernel(t):
    return pl.pallas_call(
        _mask_body,
        grid=(_N // _MB,),
        in_specs=[pl.BlockSpec((_MB,), lambda i: (i,))],
        out_specs=pl.BlockSpec((_MB, _MAX_PARAMS), lambda i: (i, 0)),
        out_shape=jax.ShapeDtypeStruct((_N, _MAX_PARAMS), jnp.float32),
    )(t)


def _gelu_exact(v):
    return 0.5 * v * (1.0 + lax.erf(v * np.float32(1.0 / np.sqrt(2.0))))


def _mlp_body(be_ref, xs_ref, w1_ref, b1_ref, w2_ref, b2_ref, w3_ref, b3_ref,
              out_ref):
    x = xs_ref[...].astype(jnp.bfloat16)
    h = jnp.dot(x, w1_ref[0], preferred_element_type=jnp.float32) + b1_ref[0]
    h = _gelu_exact(h)
    h = jnp.dot(h, w2_ref[0], preferred_element_type=jnp.float32) + b2_ref[0]
    h = _gelu_exact(h)
    p = jnp.dot(h, w3_ref[0], preferred_element_type=jnp.float32) + b3_ref[0]
    out_ref[:, :_MAX_PARAMS] = p


def _grouped_mlp(block_expert, xs, W1b, b1r, W2, b2r, W3, b3r):
    grid_spec = pltpu.PrefetchScalarGridSpec(
        num_scalar_prefetch=1,
        grid=(_GH,),
        in_specs=[
            pl.BlockSpec((_B, _IN), lambda i, be: (i, 0)),
            pl.BlockSpec((1, _IN, _H1), lambda i, be: (be[i], 0, 0)),
            pl.BlockSpec((1, 1, _H1), lambda i, be: (be[i], 0, 0)),
            pl.BlockSpec((1, _H1, _H2), lambda i, be: (be[i], 0, 0)),
            pl.BlockSpec((1, 1, _H2), lambda i, be: (be[i], 0, 0)),
            pl.BlockSpec((1, _H2, _MAX_PARAMS), lambda i, be: (be[i], 0, 0)),
            pl.BlockSpec((1, 1, _MAX_PARAMS), lambda i, be: (be[i], 0, 0)),
        ],
        out_specs=pl.BlockSpec((_B, _LANES), lambda i, be: (i, 0)),
    )
    return pl.pallas_call(
        _mlp_body,
        grid_spec=grid_spec,
        out_shape=jax.ShapeDtypeStruct((_MPH, _LANES), jnp.float32),
        compiler_params=pltpu.CompilerParams(
            dimension_semantics=("arbitrary",),
        ),
    )(block_expert, xs, W1b, b1r, W2, b2r, W3, b3r)


def kernel(x, geometry_types, W1, b1, W2, b2, W3, b3):
    t = geometry_types.astype(jnp.int32)
    W1b = W1.astype(jnp.bfloat16)
    b1r = b1.reshape(_NUM_TYPES, 1, _H1)
    b2r = b2.reshape(_NUM_TYPES, 1, _H2)
    b3r = b3.reshape(_NUM_TYPES, 1, _MAX_PARAMS)

    halves = []
    for h in range(_NH):
        th = lax.slice(t, (h * _NT,), ((h + 1) * _NT,))
        pos2, be2 = _route_kernel(th.reshape(_TR, _TCOLS))
        pos = pos2.reshape(_NT)
        xs = _make_dispatch(h)(x, pos)
        ps = _grouped_mlp(be2[0], xs, W1b, b1r, W2, b2r, W3, b3r)
        halves.append(_make_collect()(ps, pos))

    params = jnp.concatenate(
        [p[:, :_MAX_PARAMS] for p in halves], axis=0)
    masks = _mask_kernel(t)
    return params, masks
